# Initial kernel scaffold; baseline (speedup 1.0000x reference)
import jax
import jax.numpy as jnp
from jax.experimental import pallas as pl


def kernel(x, table):
    raise NotImplementedError("write your pallas kernel here")


# speedup vs baseline: 1.2821x; 1.2821x over previous
import functools
import math

import jax
import jax.numpy as jnp
from jax import lax
from jax.experimental import pallas as pl
from jax.experimental.pallas import tpu as pltpu
from jax.experimental.pallas import tpu_sc as plsc

_GRP = 128
_LANE = 16


@functools.lru_cache(maxsize=None)
def _make_sc_gather(B, V, D, K):
    info = plsc.get_sparse_core_info()
    nc, ns = info.num_cores, info.num_subcores
    nw = nc * ns
    groups = B // _GRP
    g_per_w = groups // nw
    n_chunk = g_per_w // K
    C = K * _GRP
    assert groups % nw == 0 and g_per_w % K == 0
    scale = math.sqrt(D)
    mesh = plsc.VectorSubcoreMesh(core_axis_name="c", subcore_axis_name="s")

    @functools.partial(
        pl.kernel,
        mesh=mesh,
        compiler_params=pltpu.CompilerParams(use_tc_tiling_on_sc=False),
        out_type=jax.ShapeDtypeStruct((B, D), jnp.float32),
        scratch_types=[
            pltpu.VMEM((K, _GRP), jnp.int32),
            pltpu.VMEM((C, D), jnp.float32),
            pltpu.SemaphoreType.DMA,
        ],
    )
    def sc_gather(idx_hbm, table_hbm, out_hbm, idx_v, rows_v, sem):
        wid = lax.axis_index("s") * nc + lax.axis_index("c")
        gbase = wid * g_per_w

        def chunk_body(g, carry):
            goff = gbase + g * K
            pltpu.sync_copy(idx_hbm.at[pl.ds(goff, K)], idx_v)
            copies = []
            for j in range(K):
                copies.append(
                    pltpu.async_copy(
                        table_hbm.at[idx_v.at[j]],
                        rows_v.at[pl.ds(j * _GRP, _GRP)],
                        sem,
                    )
                )
            for cp in copies:
                cp.wait()

            def row_body(r, c2):
                for h in range(0, D, _LANE):
                    rows_v[r, pl.ds(h, _LANE)] = (
                        rows_v[r, pl.ds(h, _LANE)] * scale
                    )
                return c2

            lax.fori_loop(0, C, row_body, 0)
            pltpu.sync_copy(rows_v, out_hbm.at[pl.ds(goff * _GRP, C)])
            return carry

        lax.fori_loop(0, n_chunk, chunk_body, 0)

    return sc_gather


def kernel(x, table):
    bt, f = x.shape
    v, d = table.shape
    b = bt * f
    idx = x.reshape(b // _GRP, _GRP).astype(jnp.int32)
    out = _make_sc_gather(b, v, d, 8)(idx, table)
    return out.reshape(bt, f, d)


# speedup vs baseline: 1.3534x; 1.0556x over previous
import functools
import math

import jax
import jax.numpy as jnp
from jax import lax
from jax.experimental import pallas as pl
from jax.experimental.pallas import tpu as pltpu
from jax.experimental.pallas import tpu_sc as plsc

_GRP = 128
_LANE = 16


@functools.lru_cache(maxsize=None)
def _make_sc_gather(B, V, D, K):
    info = plsc.get_sparse_core_info()
    nc, ns = info.num_cores, info.num_subcores
    nw = nc * ns
    groups = B // _GRP
    g_per_w = groups // nw
    n_chunk = g_per_w // K
    C = K * _GRP
    assert groups % nw == 0 and g_per_w % K == 0
    scale = math.sqrt(D)
    mesh = plsc.VectorSubcoreMesh(core_axis_name="c", subcore_axis_name="s")

    @functools.partial(
        pl.kernel,
        mesh=mesh,
        compiler_params=pltpu.CompilerParams(use_tc_tiling_on_sc=False),
        out_type=jax.ShapeDtypeStruct((B, D), jnp.float32),
        scratch_types=[
            pltpu.VMEM((K, _GRP), jnp.int32),
            pltpu.VMEM((C, D), jnp.float32),
            pltpu.SemaphoreType.DMA,
        ],
    )
    def sc_gather(idx_hbm, table_hbm, out_hbm, idx_v, rows_v, sem):
        wid = lax.axis_index("s") * nc + lax.axis_index("c")
        gbase = wid * g_per_w

        def chunk_body(g, carry):
            goff = gbase + g * K
            pltpu.sync_copy(idx_hbm.at[pl.ds(goff, K)], idx_v)
            copies = []
            for j in range(K):
                copies.append(
                    pltpu.async_copy(
                        table_hbm.at[idx_v.at[j]],
                        rows_v.at[pl.ds(j * _GRP, _GRP)],
                        sem,
                    )
                )
            for cp in copies:
                cp.wait()

            def row_body(r, c2):
                for h in range(0, D, _LANE):
                    rows_v[r, pl.ds(h, _LANE)] = (
                        rows_v[r, pl.ds(h, _LANE)] * scale
                    )
                return c2

            lax.fori_loop(0, C, row_body, 0, unroll=8)
            pltpu.sync_copy(rows_v, out_hbm.at[pl.ds(goff * _GRP, C)])
            return carry

        lax.fori_loop(0, n_chunk, chunk_body, 0)

    return sc_gather


def kernel(x, table):
    bt, f = x.shape
    v, d = table.shape
    b = bt * f
    idx = x.reshape(b // _GRP, _GRP).astype(jnp.int32)
    out = _make_sc_gather(b, v, d, 8)(idx, table)
    return out.reshape(bt, f, d)


# speedup vs baseline: 1.3781x; 1.0183x over previous
import functools
import math

import jax
import jax.numpy as jnp
from jax import lax
from jax.experimental import pallas as pl
from jax.experimental.pallas import tpu as pltpu
from jax.experimental.pallas import tpu_sc as plsc

_GRP = 128
_LANE = 16


@functools.lru_cache(maxsize=None)
def _make_sc_gather(B, V, D, K, S):
    info = plsc.get_sparse_core_info()
    nc, ns = info.num_cores, info.num_subcores
    nw = nc * ns
    groups = B // _GRP
    g_per_w = groups // nw
    n_chunk = g_per_w // K
    C = K * _GRP
    assert groups % nw == 0 and g_per_w % K == 0 and n_chunk % S == 0
    scale = math.sqrt(D)
    mesh = plsc.VectorSubcoreMesh(core_axis_name="c", subcore_axis_name="s")

    @functools.partial(
        pl.kernel,
        mesh=mesh,
        compiler_params=pltpu.CompilerParams(use_tc_tiling_on_sc=False),
        out_type=jax.ShapeDtypeStruct((B, D), jnp.float32),
        scratch_types=[
            pltpu.VMEM((S, K, _GRP), jnp.int32),
            pltpu.VMEM((S, C, D), jnp.float32),
        ]
        + [pltpu.SemaphoreType.DMA] * (2 * S),
    )
    def sc_gather(idx_hbm, table_hbm, out_hbm, idx_v, rows_v, *sems):
        sem_g = sems[:S]
        sem_w = sems[S:]
        wid = lax.axis_index("s") * nc + lax.axis_index("c")
        gbase = wid * g_per_w

        def fetch_and_fire(b, c):
            goff = gbase + c * K
            pltpu.sync_copy(idx_hbm.at[pl.ds(goff, K)], idx_v.at[b])
            for j in range(K):
                pltpu.async_copy(
                    table_hbm.at[idx_v.at[b, j]],
                    rows_v.at[b, pl.ds(j * _GRP, _GRP)],
                    sem_g[b],
                )

        def drain_gather(b, c):
            pltpu.make_async_copy(
                out_hbm.at[pl.ds(0, C)], rows_v.at[b], sem_g[b]
            ).wait()

        def wait_write(b):
            pltpu.make_async_copy(
                rows_v.at[b], out_hbm.at[pl.ds(0, C)], sem_w[b]
            ).wait()

        for b in range(S):
            fetch_and_fire(b, b)

        def outer(g, carry):
            for b in range(S):
                c = g * S + b
                drain_gather(b, c)

                def row_body(r, c2):
                    for h in range(0, D, _LANE):
                        rows_v[b, r, pl.ds(h, _LANE)] = (
                            rows_v[b, r, pl.ds(h, _LANE)] * scale
                        )
                    return c2

                lax.fori_loop(0, C, row_body, 0, unroll=8)
                pltpu.async_copy(
                    rows_v.at[b],
                    out_hbm.at[pl.ds((gbase + c * K) * _GRP, C)],
                    sem_w[b],
                )
                bp = (b - 1) % S
                t = c + S - 1
                ok = jnp.logical_and(c >= 1, t <= n_chunk - 1)

                @pl.when(ok)
                def _():
                    wait_write(bp)
                    fetch_and_fire(bp, t)

            return carry

        lax.fori_loop(0, n_chunk // S, outer, 0)
        for b in range(S):
            wait_write(b)

    return sc_gather


def kernel(x, table):
    bt, f = x.shape
    v, d = table.shape
    b = bt * f
    idx = x.reshape(b // _GRP, _GRP).astype(jnp.int32)
    out = _make_sc_gather(b, v, d, 2, 4)(idx, table)
    return out.reshape(bt, f, d)
